# Initial kernel scaffold; baseline (speedup 1.0000x reference)
import jax
import jax.numpy as jnp
from jax.experimental import pallas as pl


def kernel(output, target):
    raise NotImplementedError("write your pallas kernel here")


# speedup vs baseline: 2.4799x; 2.4799x over previous
import functools
import math

import jax
import jax.numpy as jnp
from jax.experimental import pallas as pl

_SIZE = 100000
_PADDING_IDX = 0
_LABEL_SMOOTHING = 0.1
_SV = _LABEL_SMOOTHING / (_SIZE - 2)
_CONF = 1.0 - _LABEL_SMOOTHING
_C0 = (_SIZE - 2) * _SV * math.log(_SV) + _CONF * math.log(_CONF)

_BLOCK_W = 2048


def _kl_kernel(x_ref, t_ref, out_ref):
    k = pl.program_id(0)
    x = x_ref[...]
    n, bw = x.shape
    cols = k * _BLOCK_W + jax.lax.broadcasted_iota(jnp.int32, (n, bw), 1)
    t = t_ref[...]
    valid = (cols < _SIZE) & (cols != _PADDING_IDX)
    coef = jnp.where(cols == t, -_CONF, -_SV)
    val = jnp.where(valid, x, 0.0) * coef
    row = jnp.sum(val, axis=1, keepdims=True)
    mask = (t != _PADDING_IDX).astype(jnp.float32)
    partial = jnp.sum(row * mask, axis=(0, 1), keepdims=True)

    @pl.when(k == 0)
    def _init():
        out_ref[...] = _C0 * jnp.sum(mask, axis=(0, 1), keepdims=True) + partial

    @pl.when(k != 0)
    def _acc():
        out_ref[...] += partial


@functools.partial(jax.jit, static_argnames=("interpret",))
def kernel(output, target, interpret=False):
    n = output.shape[0]
    t = target.astype(jnp.int32)
    num_blocks = pl.cdiv(_SIZE, _BLOCK_W)
    out = pl.pallas_call(
        _kl_kernel,
        grid=(num_blocks,),
        in_specs=[
            pl.BlockSpec((n, _BLOCK_W), lambda k: (0, k)),
            pl.BlockSpec((n, 1), lambda k: (0, 0)),
        ],
        out_specs=pl.BlockSpec((1, 1), lambda k: (0, 0)),
        out_shape=jax.ShapeDtypeStruct((1, 1), jnp.float32),
        interpret=interpret,
    )(output, t)
    return out[0, 0]
